# Initial kernel scaffold; baseline (speedup 1.0000x reference)
import jax
import jax.numpy as jnp
from jax.experimental import pallas as pl


def kernel(points_q, points, values):
    raise NotImplementedError("write your pallas kernel here")


# speedup vs baseline: 1.1765x; 1.1765x over previous
import functools

import jax
import jax.numpy as jnp
from jax import lax
from jax.experimental import pallas as pl
from jax.experimental.pallas import tpu as pltpu
from jax.experimental.pallas import tpu_sc as plsc

Q = 1024
D = 128
K = 100000
V = 64
BK = 2000
NB = K // BK


def _nn_body(qt_ref, p_ref, idx_ref, minval_ref):
    b = pl.program_id(0)

    @pl.when(b == 0)
    def _init():
        minval_ref[...] = jnp.full((1, Q), jnp.inf, dtype=jnp.float32)
        idx_ref[...] = jnp.zeros((1, Q), dtype=jnp.int32)

    qt = qt_ref[...]
    p = p_ref[0]
    psq = jnp.sum(p * p, axis=1, keepdims=True)
    mm = lax.dot_general(p, qt, (((1,), (0,)), ((), ())),
                         preferred_element_type=jnp.float32)
    dist = psq - 2.0 * mm

    bmin = jnp.min(dist, axis=0, keepdims=True)
    rows = lax.broadcasted_iota(jnp.int32, (BK, Q), 0)
    bidx = jnp.min(jnp.where(dist == bmin, rows, K), axis=0,
                   keepdims=True)

    run = minval_ref[...]
    better = bmin < run
    minval_ref[...] = jnp.where(better, bmin, run)
    idx_ref[...] = jnp.where(better, b * BK + bidx, idx_ref[...])


def _nn_argmin(points_qt, points3):
    return pl.pallas_call(
        _nn_body,
        grid=(NB,),
        in_specs=[
            pl.BlockSpec((D, Q), lambda b: (0, 0)),
            pl.BlockSpec((1, BK, D), lambda b: (b, 0, 0)),
        ],
        out_specs=pl.BlockSpec((1, Q), lambda b: (0, 0)),
        out_shape=jax.ShapeDtypeStruct((1, Q), jnp.int32),
        scratch_shapes=[pltpu.VMEM((1, Q), jnp.float32)],
        compiler_params=pltpu.CompilerParams(
            dimension_semantics=("arbitrary",),
        ),
    )(points_qt, points3)


_NC, _NS = 2, 16
_NW = _NC * _NS
_BPW = Q // _NW


@functools.cache
def _make_sc_gather():
    @functools.partial(
        pl.kernel,
        mesh=plsc.VectorSubcoreMesh(core_axis_name="c", subcore_axis_name="s"),
        out_type=jax.ShapeDtypeStruct((Q, V), jnp.float32),
        scratch_types=[
            pltpu.VMEM((_BPW,), jnp.int32),
            pltpu.VMEM((_BPW, V), jnp.float32),
            pltpu.SemaphoreType.DMA,
        ],
        compiler_params=pltpu.CompilerParams(use_tc_tiling_on_sc=False),
    )
    def _sc_gather(values_hbm, idx_hbm, out_hbm, idx_v, rows_v, sem):
        wid = lax.axis_index("s") * _NC + lax.axis_index("c")
        base = wid * _BPW
        pltpu.sync_copy(idx_hbm.at[pl.ds(base, _BPW)], idx_v)
        pltpu.async_copy(values_hbm.at[idx_v], rows_v, sem).wait()
        pltpu.sync_copy(rows_v, out_hbm.at[pl.ds(base, _BPW)])

    return _sc_gather


def kernel(points_q, points, values):
    points_qt = points_q.T
    points3 = points.reshape(NB, BK, D)
    nn_idx = _nn_argmin(points_qt, points3).reshape(Q)
    return _make_sc_gather()(values, nn_idx)


# speedup vs baseline: 2.0319x; 1.7271x over previous
import functools

import jax
import jax.numpy as jnp
from jax import lax
from jax.experimental import pallas as pl
from jax.experimental.pallas import tpu as pltpu
from jax.experimental.pallas import tpu_sc as plsc

Q = 1024
D = 128
K = 100000
V = 64
BK = 2000
NB = K // BK

TH = 8
CH = 400
NCH = BK // CH
NSL = CH // TH

VB = 2048
NVB = (K + VB - 1) // VB


def _nn_body(qt_ref, p_ref, vt_ref, idx_ref, v2_ref, m_ref, i_ref):
    b = pl.program_id(0)

    @pl.when(b == 0)
    def _init():
        m_ref[...] = jnp.full((TH, Q), jnp.inf, dtype=jnp.float32)
        i_ref[...] = jnp.zeros((TH, Q), dtype=jnp.int32)

    @pl.when(b < NVB)
    def _retile():
        v2_ref[0, :, 0:V] = jnp.transpose(vt_ref[...])
        v2_ref[0, :, V:2 * V] = jnp.zeros((VB, V), dtype=jnp.float32)

    qt2 = -2.0 * qt_ref[...]
    m = m_ref[...]
    i = i_ref[...]

    for c in range(NCH):
        p = p_ref[0, c * CH:(c + 1) * CH, :]
        psq = jnp.sum(p * p, axis=1, keepdims=True)
        mm = lax.dot_general(p, qt2, (((1,), (0,)), ((), ())),
                             preferred_element_type=jnp.float32)
        for t in range(NSL):
            d = (lax.slice(mm, (t * TH, 0), ((t + 1) * TH, Q))
                 + lax.slice(psq, (t * TH, 0), ((t + 1) * TH, 1)))
            sid = b * (BK // TH) + c * NSL + t
            lt = d < m
            m = jnp.where(lt, d, m)
            i = jnp.where(lt, sid, i)

    m_ref[...] = m
    i_ref[...] = i

    @pl.when(b == NB - 1)
    def _fin():
        mm_ = m_ref[...]
        kk_ = i_ref[...] * TH + lax.broadcasted_iota(jnp.int32, (TH, Q), 0)
        h = TH
        while h > 1:
            h //= 2
            ma = lax.slice(mm_, (0, 0), (h, Q))
            mb = lax.slice(mm_, (h, 0), (2 * h, Q))
            ka = lax.slice(kk_, (0, 0), (h, Q))
            kb = lax.slice(kk_, (h, 0), (2 * h, Q))
            takeb = (mb < ma) | ((mb == ma) & (kb < ka))
            mm_ = jnp.where(takeb, mb, ma)
            kk_ = jnp.where(takeb, kb, ka)
        idx_ref[...] = kk_


def _nn_argmin(points_qt, points3, values_t):
    return pl.pallas_call(
        _nn_body,
        grid=(NB,),
        in_specs=[
            pl.BlockSpec((D, Q), lambda b: (0, 0)),
            pl.BlockSpec((1, BK, D), lambda b: (b, 0, 0)),
            pl.BlockSpec((V, VB), lambda b: (0, jnp.minimum(b, NVB - 1))),
        ],
        out_specs=[
            pl.BlockSpec((1, Q), lambda b: (0, 0)),
            pl.BlockSpec((1, VB, 2 * V),
                         lambda b: (jnp.minimum(b, NVB - 1), 0, 0)),
        ],
        out_shape=[
            jax.ShapeDtypeStruct((1, Q), jnp.int32),
            jax.ShapeDtypeStruct((NVB, VB, 2 * V), jnp.float32),
        ],
        scratch_shapes=[pltpu.VMEM((TH, Q), jnp.float32),
                        pltpu.VMEM((TH, Q), jnp.int32)],
        compiler_params=pltpu.CompilerParams(
            dimension_semantics=("arbitrary",),
        ),
    )(points_qt, points3, values_t)


_NC, _NS = 2, 16
_NW = _NC * _NS
_BPW = Q // _NW


@functools.cache
def _make_sc_gather():
    @functools.partial(
        pl.kernel,
        mesh=plsc.VectorSubcoreMesh(core_axis_name="c", subcore_axis_name="s"),
        out_type=jax.ShapeDtypeStruct((Q, V), jnp.float32),
        scratch_types=[
            pltpu.VMEM((_BPW,), jnp.int32),
            pltpu.VMEM((_BPW, 2 * V), jnp.float32),
            pltpu.VMEM((_BPW, V), jnp.float32),
            pltpu.SemaphoreType.DMA,
        ],
    )
    def _sc_gather(table_hbm, idx_hbm, out_hbm, idx_v, rows_v, out_v, sem):
        wid = lax.axis_index("s") * _NC + lax.axis_index("c")
        base = wid * _BPW
        pltpu.sync_copy(idx_hbm.at[pl.ds(base, _BPW)], idx_v)
        pltpu.async_copy(table_hbm.at[idx_v], rows_v, sem).wait()
        for i in range(_BPW):
            for j in range(V // 16):
                out_v[i, pl.ds(16 * j, 16)] = rows_v[i, pl.ds(16 * j, 16)]
        pltpu.sync_copy(out_v, out_hbm.at[pl.ds(base, _BPW)])

    return _sc_gather


def kernel(points_q, points, values):
    points_qt = points_q.T
    points3 = points.reshape(NB, BK, D)
    values_t = values.T
    nn_idx, table = _nn_argmin(points_qt, points3, values_t)
    return _make_sc_gather()(table.reshape(NVB * VB, 2 * V),
                             nn_idx.reshape(Q))
